# baseline (speedup 1.0000x reference)
import jax
import jax.numpy as jnp
from jax.experimental import pallas as pl

N = 100000
G = 8
RN = N // G
BR = 1600


def _mlp_kernel(x_ref, w1_ref, b1_ref, w2_ref, b2_ref, ws_ref, bs_ref,
                we_ref, be_ref, wm_ref, bm_ref, seg_ref, emb_ref, mar_ref):
    xb = x_ref[...]
    h = jnp.maximum(
        jnp.dot(xb, w1_ref[...], preferred_element_type=jnp.float32)
        + b1_ref[...], 0.0)
    g = jnp.maximum(
        jnp.dot(h, w2_ref[...], preferred_element_type=jnp.float32)
        + b2_ref[...], 0.0)
    seg_ref[...] = (jnp.dot(g, ws_ref[...], preferred_element_type=jnp.float32)
                    + bs_ref[...])
    emb_ref[...] = (jnp.dot(g, we_ref[...], preferred_element_type=jnp.float32)
                    + be_ref[...])
    mar_ref[...] = (jnp.dot(g, wm_ref[...], preferred_element_type=jnp.float32)
                    + bm_ref[...])


def kernel(x, frag_ids, group_ids, edge_index1, edge_index2, params):
    p = params
    eye = jnp.eye(G, dtype=jnp.float32)

    w1 = jnp.kron(eye, p["Wb"])
    b1 = jnp.tile(p["bb"], G).reshape(1, -1)
    w2v = jnp.concatenate([p["Ws"], p["Wi"]], axis=1)
    w2 = jnp.kron(eye, w2v)
    b2 = jnp.tile(jnp.concatenate([p["bs"], p["bi"]]), G).reshape(1, -1)

    z = jnp.zeros((16, 5), jnp.float32)
    wsv = jnp.concatenate([p["Wcls"], z], axis=0)
    ws = jnp.kron(eye, wsv)
    bs = jnp.tile(p["bcls"], G).reshape(1, -1)

    z3 = jnp.zeros((16, 3), jnp.float32)
    wev = jnp.concatenate([z3, p["Wemb"][:, :3]], axis=0)
    we = jnp.kron(eye, wev)
    be = jnp.tile(p["bemb"][:3], G).reshape(1, -1)

    z1 = jnp.zeros((16, 1), jnp.float32)
    wmv = jnp.concatenate([z1, p["Wemb"][:, 3:4]], axis=0)
    wm = jnp.kron(eye, wmv)
    bm = jnp.tile(p["bemb"][3:], G).reshape(1, -1)

    xr = x.reshape(RN, 5 * G)

    def rows(i):
        return (i, 0)

    def whole(i):
        return (0, 0)

    nblk = (RN + BR - 1) // BR
    seg, emb, mar = pl.pallas_call(
        _mlp_kernel,
        grid=(nblk,),
        in_specs=[pl.BlockSpec((BR, 5 * G), rows),
                  pl.BlockSpec(w1.shape, whole), pl.BlockSpec(b1.shape, whole),
                  pl.BlockSpec(w2.shape, whole), pl.BlockSpec(b2.shape, whole),
                  pl.BlockSpec(ws.shape, whole), pl.BlockSpec(bs.shape, whole),
                  pl.BlockSpec(we.shape, whole), pl.BlockSpec(be.shape, whole),
                  pl.BlockSpec(wm.shape, whole), pl.BlockSpec(bm.shape, whole)],
        out_specs=[pl.BlockSpec((BR, 5 * G), rows),
                   pl.BlockSpec((BR, 3 * G), rows),
                   pl.BlockSpec((BR, G), rows)],
        out_shape=[jax.ShapeDtypeStruct((RN, 5 * G), jnp.float32),
                   jax.ShapeDtypeStruct((RN, 3 * G), jnp.float32),
                   jax.ShapeDtypeStruct((RN, G), jnp.float32)],
    )(xr, w1, b1, w2, b2, ws, bs, we, be, wm, bm)
    return (seg.reshape(N, 5), emb.reshape(N, 3), mar.reshape(N, 1))


# speedup vs baseline: 10.8251x; 10.8251x over previous
import jax
import jax.numpy as jnp
from jax.experimental import pallas as pl

N = 100000
BL = 12800


def _dgt(a, b):
    return jax.lax.dot_general(a, b, (((0,), (0,)), ((), ())),
                               preferred_element_type=jnp.float32)


def _mlp_kernel(x_ref, w1_ref, ws_ref, wi_ref, wc_ref, we_ref,
                seg_ref, emb_ref, mar_ref):
    xb = x_ref[...]
    ones = jnp.ones((1, xb.shape[1]), jnp.float32)
    x6 = jnp.concatenate([xb, ones], axis=0)
    h = jnp.maximum(_dgt(w1_ref[...], x6), 0.0)
    h33 = jnp.concatenate([h, ones], axis=0)
    sf = jnp.maximum(_dgt(ws_ref[...], h33), 0.0)
    inf_ = jnp.maximum(_dgt(wi_ref[...], h33), 0.0)
    sf17 = jnp.concatenate([sf, ones], axis=0)
    if17 = jnp.concatenate([inf_, ones], axis=0)
    seg_ref[...] = _dgt(wc_ref[...], sf17)
    e4 = _dgt(we_ref[...], if17)
    emb_ref[...] = e4[:3]
    mar_ref[...] = e4[3:4]


def kernel(x, frag_ids, group_ids, edge_index1, edge_index2, params):
    p = params
    w1 = jnp.concatenate([p["Wb"], p["bb"][None, :]], axis=0)
    ws = jnp.concatenate([p["Ws"], p["bs"][None, :]], axis=0)
    wi = jnp.concatenate([p["Wi"], p["bi"][None, :]], axis=0)
    wc = jnp.concatenate([p["Wcls"], p["bcls"][None, :]], axis=0)
    we = jnp.concatenate([p["Wemb"], p["bemb"][None, :]], axis=0)

    xt = x.T

    def lanes(i):
        return (0, i)

    def whole(i):
        return (0, 0)

    nblk = (N + BL - 1) // BL
    segt, embt, mart = pl.pallas_call(
        _mlp_kernel,
        grid=(nblk,),
        in_specs=[pl.BlockSpec((5, BL), lanes),
                  pl.BlockSpec(w1.shape, whole), pl.BlockSpec(ws.shape, whole),
                  pl.BlockSpec(wi.shape, whole), pl.BlockSpec(wc.shape, whole),
                  pl.BlockSpec(we.shape, whole)],
        out_specs=[pl.BlockSpec((5, BL), lanes),
                   pl.BlockSpec((3, BL), lanes),
                   pl.BlockSpec((1, BL), lanes)],
        out_shape=[jax.ShapeDtypeStruct((5, N), jnp.float32),
                   jax.ShapeDtypeStruct((3, N), jnp.float32),
                   jax.ShapeDtypeStruct((1, N), jnp.float32)],
    )(xt, w1, ws, wi, wc, we)
    return (segt.T, embt.T, mart.T)


# speedup vs baseline: 23.9327x; 2.2109x over previous
import jax
import jax.numpy as jnp
from jax.experimental import pallas as pl

N = 100000
BL = 25600


def _mlp_kernel(x_ref, w1_ref, b1_ref, ws_ref, wi_ref, bs_ref, bi_ref,
                wc_ref, bc_ref, we_ref, be_ref, seg_ref, emb_ref, mar_ref):
    xb = x_ref[...]
    b1 = b1_ref[...].T
    h = jnp.maximum(
        jax.lax.dot_general(w1_ref[...], xb, (((0,), (0,)), ((), ())),
                            preferred_element_type=jnp.float32)
        + b1, 0.0)
    w2 = jnp.concatenate([ws_ref[...], wi_ref[...]], axis=0)
    b2 = jnp.concatenate([bs_ref[...].T, bi_ref[...].T], axis=0)
    g = jnp.maximum(
        jnp.dot(w2, h, preferred_element_type=jnp.float32) + b2, 0.0)
    seg_ref[...] = (
        jnp.dot(wc_ref[...], g[:16], preferred_element_type=jnp.float32)
        + bc_ref[...].T)
    e4 = (jnp.dot(we_ref[...], g[16:32], preferred_element_type=jnp.float32)
          + be_ref[...].T)
    emb_ref[...] = e4[:3]
    mar_ref[...] = e4[3:4]


def kernel(x, frag_ids, group_ids, edge_index1, edge_index2, params):
    p = params
    xt = x.T
    w1 = p["Wb"]
    ws = p["Ws"].T
    wi = p["Wi"].T
    wc = p["Wcls"].T
    we = p["Wemb"].T
    b1 = p["bb"][None, :]
    bs = p["bs"][None, :]
    bi = p["bi"][None, :]
    bc = p["bcls"][None, :]
    be = p["bemb"][None, :]

    def lanes(i):
        return (0, i)

    def whole(i):
        return (0, 0)

    nblk = (N + BL - 1) // BL
    segt, embt, mart = pl.pallas_call(
        _mlp_kernel,
        grid=(nblk,),
        in_specs=[pl.BlockSpec((5, BL), lanes),
                  pl.BlockSpec(w1.shape, whole), pl.BlockSpec(b1.shape, whole),
                  pl.BlockSpec(ws.shape, whole), pl.BlockSpec(wi.shape, whole),
                  pl.BlockSpec(bs.shape, whole), pl.BlockSpec(bi.shape, whole),
                  pl.BlockSpec(wc.shape, whole), pl.BlockSpec(bc.shape, whole),
                  pl.BlockSpec(we.shape, whole), pl.BlockSpec(be.shape, whole)],
        out_specs=[pl.BlockSpec((5, BL), lanes),
                   pl.BlockSpec((3, BL), lanes),
                   pl.BlockSpec((1, BL), lanes)],
        out_shape=[jax.ShapeDtypeStruct((5, N), jnp.float32),
                   jax.ShapeDtypeStruct((3, N), jnp.float32),
                   jax.ShapeDtypeStruct((1, N), jnp.float32)],
    )(xt, w1, b1, ws, wi, bs, bi, wc, bc, we, be)
    return (segt.T, embt.T, mart.T)


# speedup vs baseline: 24.4196x; 1.0203x over previous
import jax
import jax.numpy as jnp
from jax.experimental import pallas as pl

N = 100000
BL = 51200


def _mlp_kernel(x_ref, w1_ref, b1_ref, ws_ref, wi_ref, bs_ref, bi_ref,
                wc_ref, bc_ref, we_ref, be_ref, seg_ref, emb_ref, mar_ref):
    xb = x_ref[...]
    b1 = b1_ref[...].T
    h = jnp.maximum(
        jax.lax.dot_general(w1_ref[...], xb, (((0,), (0,)), ((), ())),
                            preferred_element_type=jnp.float32)
        + b1, 0.0)
    w2 = jnp.concatenate([ws_ref[...], wi_ref[...]], axis=0)
    b2 = jnp.concatenate([bs_ref[...].T, bi_ref[...].T], axis=0)
    g = jnp.maximum(
        jnp.dot(w2, h, preferred_element_type=jnp.float32) + b2, 0.0)
    seg_ref[...] = (
        jnp.dot(wc_ref[...], g[:16], preferred_element_type=jnp.float32)
        + bc_ref[...].T)
    e4 = (jnp.dot(we_ref[...], g[16:32], preferred_element_type=jnp.float32)
          + be_ref[...].T)
    emb_ref[...] = e4[:3]
    mar_ref[...] = e4[3:4]


def kernel(x, frag_ids, group_ids, edge_index1, edge_index2, params):
    p = params
    xt = x.T
    w1 = p["Wb"]
    ws = p["Ws"].T
    wi = p["Wi"].T
    wc = p["Wcls"].T
    we = p["Wemb"].T
    b1 = p["bb"][None, :]
    bs = p["bs"][None, :]
    bi = p["bi"][None, :]
    bc = p["bcls"][None, :]
    be = p["bemb"][None, :]

    def lanes(i):
        return (0, i)

    def whole(i):
        return (0, 0)

    nblk = (N + BL - 1) // BL
    segt, embt, mart = pl.pallas_call(
        _mlp_kernel,
        grid=(nblk,),
        in_specs=[pl.BlockSpec((5, BL), lanes),
                  pl.BlockSpec(w1.shape, whole), pl.BlockSpec(b1.shape, whole),
                  pl.BlockSpec(ws.shape, whole), pl.BlockSpec(wi.shape, whole),
                  pl.BlockSpec(bs.shape, whole), pl.BlockSpec(bi.shape, whole),
                  pl.BlockSpec(wc.shape, whole), pl.BlockSpec(bc.shape, whole),
                  pl.BlockSpec(we.shape, whole), pl.BlockSpec(be.shape, whole)],
        out_specs=[pl.BlockSpec((5, BL), lanes),
                   pl.BlockSpec((3, BL), lanes),
                   pl.BlockSpec((1, BL), lanes)],
        out_shape=[jax.ShapeDtypeStruct((5, N), jnp.float32),
                   jax.ShapeDtypeStruct((3, N), jnp.float32),
                   jax.ShapeDtypeStruct((1, N), jnp.float32)],
    )(xt, w1, b1, ws, wi, bs, bi, wc, bc, we, be)
    return (segt.T, embt.T, mart.T)
